# baseline (speedup 1.0000x reference)
import functools

import jax
import jax.numpy as jnp
from jax import lax
from jax.experimental import pallas as pl
from jax.experimental.pallas import tpu as pltpu
from jax.experimental.pallas import tpu_sc as plsc

ROWS = 128
COLS = 32768
LANES = 16
NUM_WORKERS = 32
ROWS_PER_WORKER = ROWS // NUM_WORKERS
STREAMS = 16
SPAN = STREAMS * LANES
VECS = COLS // LANES
STEPS = VECS // STREAMS
UNMATCHED = 65536


def _butterfly(vec, lane, op):
    for k in (8, 4, 2, 1):
        vec = op(vec, vec.at[lane ^ k].get(mode="promise_in_bounds"))
    return vec


def _row_argmax(buf, slot, lane):
    neg_inf = jnp.full((LANES,), -jnp.inf, jnp.float32)

    def maxstep(t, vals):
        base = t * SPAN
        return tuple(
            jnp.maximum(vals[s], buf[slot, pl.ds(base + s * LANES, LANES)])
            for s in range(STREAMS)
        )

    vals = lax.fori_loop(0, STEPS, maxstep, (neg_inf,) * STREAMS)
    vals = list(vals)
    while len(vals) > 1:
        vals = [jnp.maximum(vals[i], vals[i + 1]) for i in range(0, len(vals), 2)]
    mvec = _butterfly(vals[0], lane, jnp.maximum)

    big = jnp.full((LANES,), UNMATCHED, jnp.int32)

    def findstep(t, idxs):
        i = STEPS - 1 - t
        base = i * SPAN
        return tuple(
            jnp.where(
                buf[slot, pl.ds(base + s * LANES, LANES)] == mvec, i, idxs[s]
            )
            for s in range(STREAMS)
        )

    idxs = lax.fori_loop(0, STEPS, findstep, (big,) * STREAMS)
    gidx = [idxs[s] * SPAN + (s * LANES) + lane for s in range(STREAMS)]
    while len(gidx) > 1:
        gidx = [jnp.minimum(gidx[i], gidx[i + 1]) for i in range(0, len(gidx), 2)]
    return _butterfly(gidx[0], lane, jnp.minimum)


@functools.partial(
    pl.kernel,
    out_type=jax.ShapeDtypeStruct((NUM_WORKERS, LANES), jnp.int32),
    mesh=plsc.VectorSubcoreMesh(core_axis_name="c", subcore_axis_name="s"),
    scratch_types=[
        pltpu.VMEM((2, COLS), jnp.float32),
        pltpu.VMEM((LANES,), jnp.int32),
        pltpu.SemaphoreType.DMA,
        pltpu.SemaphoreType.DMA,
    ],
)
def _argmax_sc(data_hbm, out_hbm, buf, res_ref, sem0, sem1):
    cid = lax.axis_index("c")
    sid = lax.axis_index("s")
    wid = cid * 16 + sid
    row0 = wid * ROWS_PER_WORKER
    sems = (sem0, sem1)

    handles = [None, None]
    handles[0] = pltpu.async_copy(data_hbm.at[row0], buf.at[0], sems[0])

    lane = lax.iota(jnp.int32, LANES)
    resvec = jnp.zeros((LANES,), jnp.int32)
    for j in range(ROWS_PER_WORKER):
        slot = j % 2
        if j + 1 < ROWS_PER_WORKER:
            nslot = (j + 1) % 2
            handles[nslot] = pltpu.async_copy(
                data_hbm.at[row0 + j + 1], buf.at[nslot], sems[nslot]
            )
        handles[slot].wait()
        res = _row_argmax(buf, slot, lane)
        resvec = jnp.where(lane == j, res, resvec)

    res_ref[...] = resvec
    pltpu.sync_copy(res_ref, out_hbm.at[wid])


def kernel(data):
    out2 = _argmax_sc(data)
    return out2[:, :ROWS_PER_WORKER].reshape(ROWS)


# speedup vs baseline: 1.1761x; 1.1761x over previous
import functools

import jax
import jax.numpy as jnp
from jax import lax
from jax.experimental import pallas as pl
from jax.experimental.pallas import tpu as pltpu
from jax.experimental.pallas import tpu_sc as plsc

ROWS = 128
COLS = 32768
SC_ROWS = 64
TC_ROWS = ROWS - SC_ROWS
LANES = 16
NUM_WORKERS = 32
ROWS_PER_WORKER = SC_ROWS // NUM_WORKERS
STREAMS = 8
SPAN = STREAMS * LANES
VECS = COLS // LANES
STEPS = VECS // STREAMS
INT_MAX = 2**31 - 1

TC_BLOCK = 2048
TC_STEPS = COLS // TC_BLOCK


def _row_argmax(buf, slot, lane):
    neg_inf = jnp.full((LANES,), -jnp.inf, jnp.float32)
    zeros = jnp.zeros((LANES,), jnp.int32)
    init = tuple([neg_inf] * STREAMS + [zeros] * STREAMS)

    def step(t, carry):
        vals = carry[:STREAMS]
        steps = carry[STREAMS:]
        base = t * SPAN
        new_vals, new_steps = [], []
        for s in range(STREAMS):
            v = buf[slot, pl.ds(base + s * LANES, LANES)]
            c = v > vals[s]
            new_steps.append(jnp.where(c, t, steps[s]))
            new_vals.append(jnp.maximum(vals[s], v))
        return tuple(new_vals + new_steps)

    carry = lax.fori_loop(0, STEPS, step, init)
    vals = carry[:STREAMS]
    steps = carry[STREAMS:]

    pairs = [
        (vals[s], steps[s] * SPAN + (s * LANES) + lane) for s in range(STREAMS)
    ]

    def merge(a, b):
        va, ia = a
        vb, ib = b
        take_b = (vb > va) | ((vb == va) & (ib < ia))
        return (jnp.where(take_b, vb, va), jnp.where(take_b, ib, ia))

    while len(pairs) > 1:
        pairs = [merge(pairs[i], pairs[i + 1]) for i in range(0, len(pairs), 2)]
    v, idx = pairs[0]

    for k in (8, 4, 2, 1):
        perm = lane ^ k
        vb = v.at[perm].get(mode="promise_in_bounds")
        ib = idx.at[perm].get(mode="promise_in_bounds")
        v, idx = merge((v, idx), (vb, ib))
    return idx


@functools.partial(
    pl.kernel,
    out_type=jax.ShapeDtypeStruct((NUM_WORKERS, LANES), jnp.int32),
    mesh=plsc.VectorSubcoreMesh(core_axis_name="c", subcore_axis_name="s"),
    scratch_types=[
        pltpu.VMEM((2, COLS), jnp.float32),
        pltpu.VMEM((LANES,), jnp.int32),
        pltpu.SemaphoreType.DMA,
        pltpu.SemaphoreType.DMA,
    ],
)
def _argmax_sc(data_hbm, out_hbm, buf, res_ref, sem0, sem1):
    cid = lax.axis_index("c")
    sid = lax.axis_index("s")
    wid = cid * 16 + sid
    row0 = wid * ROWS_PER_WORKER
    sems = (sem0, sem1)

    handles = [None, None]
    handles[0] = pltpu.async_copy(data_hbm.at[row0], buf.at[0], sems[0])

    lane = lax.iota(jnp.int32, LANES)
    resvec = jnp.zeros((LANES,), jnp.int32)
    for j in range(ROWS_PER_WORKER):
        slot = j % 2
        if j + 1 < ROWS_PER_WORKER:
            nslot = (j + 1) % 2
            handles[nslot] = pltpu.async_copy(
                data_hbm.at[row0 + j + 1], buf.at[nslot], sems[nslot]
            )
        handles[slot].wait()
        res = _row_argmax(buf, slot, lane)
        resvec = jnp.where(lane == j, res, resvec)

    res_ref[...] = resvec
    pltpu.sync_copy(res_ref, out_hbm.at[wid])


def _tc_body(x_ref, o_ref, vmax_ref, vidx_ref):
    step = pl.program_id(0)
    x = x_ref[...]
    m = jnp.max(x, axis=1, keepdims=True)
    cols = lax.broadcasted_iota(jnp.int32, (TC_ROWS, TC_BLOCK), 1) + step * TC_BLOCK
    li = jnp.min(jnp.where(x == m, cols, INT_MAX), axis=1, keepdims=True)

    @pl.when(step == 0)
    def _():
        vmax_ref[...] = m
        vidx_ref[...] = li

    @pl.when(step > 0)
    def _():
        better = m > vmax_ref[...]
        vidx_ref[...] = jnp.where(better, li, vidx_ref[...])
        vmax_ref[...] = jnp.where(better, m, vmax_ref[...])

    @pl.when(step == TC_STEPS - 1)
    def _():
        o_ref[...] = vidx_ref[...]


_argmax_tc = pl.pallas_call(
    _tc_body,
    grid=(TC_STEPS,),
    in_specs=[
        pl.BlockSpec((TC_ROWS, TC_BLOCK), lambda i: (1, i)),
    ],
    out_specs=pl.BlockSpec((TC_ROWS, 1), lambda i: (0, 0)),
    out_shape=jax.ShapeDtypeStruct((TC_ROWS, 1), jnp.int32),
    scratch_shapes=[
        pltpu.VMEM((TC_ROWS, 1), jnp.float32),
        pltpu.VMEM((TC_ROWS, 1), jnp.int32),
    ],
)


def kernel(data):
    sc2 = _argmax_sc(data)
    tc2 = _argmax_tc(data)
    return jnp.concatenate(
        [sc2[:, :ROWS_PER_WORKER].reshape(SC_ROWS), tc2[:, 0]]
    )
